# baseline (speedup 1.0000x reference)
import functools

import jax
import jax.numpy as jnp
from jax import lax
from jax.experimental import pallas as pl
from jax.experimental.pallas import tpu as pltpu
from jax.experimental.pallas import tpu_sc as plsc

EMBED = 64
ROWS_TOTAL = 16384 * 50
NW = 32
ROWS_PER_W = ROWS_TOTAL // NW
CHUNK = 128
N_CHUNKS = ROWS_PER_W // CHUNK
K = 2
GROUP = K * CHUNK
NG = N_CHUNKS // K
NBUF = 4
N_OUTER = NG // NBUF
SCALE = 8.0

_mesh = plsc.VectorSubcoreMesh(core_axis_name="c", subcore_axis_name="s")


@functools.partial(
    pl.kernel,
    mesh=_mesh,
    compiler_params=pltpu.CompilerParams(use_tc_tiling_on_sc=False),
    out_type=jax.ShapeDtypeStruct((ROWS_TOTAL, EMBED), jnp.float32),
    scratch_types=[
        pltpu.VMEM((N_CHUNKS, CHUNK), jnp.int32),
        pltpu.VMEM((NBUF, GROUP, EMBED), jnp.float32),
    ] + [pltpu.SemaphoreType.DMA] * (2 * NBUF),
)
def _embed_gather(idx_hbm, table_hbm, out_hbm, idx_v, rows_v, *sems):
    gsem = sems[:NBUF]
    ssem = sems[NBUF:]
    wid = lax.axis_index("s") * 2 + lax.axis_index("c")
    base_row = wid * ROWS_PER_W
    base_chunk = wid * N_CHUNKS

    pltpu.sync_copy(idx_hbm.at[pl.ds(base_chunk, N_CHUNKS)], idx_v)

    def gat(g, half, j):
        return pltpu.make_async_copy(
            table_hbm.at[idx_v.at[g * K + j]],
            rows_v.at[half, pl.ds(j * CHUNK, CHUNK)],
            gsem[half])

    def sto(g, half):
        return pltpu.make_async_copy(
            rows_v.at[half],
            out_hbm.at[pl.ds(base_row + g * GROUP, GROUP)],
            ssem[half])

    def scale_half(half):
        def body(i, c):
            for rr in range(8):
                r = i * 8 + rr
                for j in range(EMBED // 16):
                    sl = pl.ds(j * 16, 16)
                    rows_v[half, r, sl] = rows_v[half, r, sl] * SCALE
            return c
        lax.fori_loop(0, GROUP // 8, body, 0)

    def step(g, b, *, store_wait, issue):
        h2 = (b + 2) % NBUF
        if store_wait:
            sto(g - 2, h2).wait()
        if issue:
            for j in range(K):
                gat(g + 2, h2, j).start()
        for j in range(K):
            gat(g, b, j).wait()
        scale_half(b)
        sto(g, b).start()

    for j in range(K):
        gat(0, 0, j).start()
    for j in range(K):
        gat(1, 1, j).start()

    step(0, 0, store_wait=False, issue=True)
    step(1, 1, store_wait=False, issue=True)
    step(2, 2, store_wait=True, issue=True)
    step(3, 3, store_wait=True, issue=True)

    def outer(t, c):
        for b in range(NBUF):
            step(t * NBUF + b, b, store_wait=True, issue=True)
        return c
    lax.fori_loop(1, N_OUTER - 1, outer, 0)

    gl = (N_OUTER - 1) * NBUF
    step(gl + 0, 0, store_wait=True, issue=True)
    step(gl + 1, 1, store_wait=True, issue=True)
    step(gl + 2, 2, store_wait=True, issue=False)
    step(gl + 3, 3, store_wait=True, issue=False)

    sto(NG - 2, 2).wait()
    sto(NG - 1, 3).wait()


def kernel(x, table):
    idx = x.reshape(NW * N_CHUNKS, CHUNK).astype(jnp.int32)
    out = _embed_gather(idx, table)
    return out.reshape(x.shape[0], x.shape[1], EMBED)
